# baseline (speedup 1.0000x reference)
import functools

import jax
import jax.numpy as jnp
from jax.experimental import pallas as pl
from jax.experimental.pallas import tpu as pltpu
from jax.experimental.pallas import tpu_sc as plsc

_N = 2048
_D = 1024
_CUT1 = 20000
_CUT2 = 40000

_SC = plsc.get_sparse_core_info()
_NC, _NS, _L = _SC.num_cores, _SC.num_subcores, _SC.num_lanes
_NW = _NC * _NS
_BPW = _N // _NW


def _proj_kernel(pt_ref, ht_ref, o0_ref, o1_ref, o2_ref):
    def part(lo, d, o_ref):
        o_ref[pl.ds(0, d), :] = jax.lax.dot_general(
            pt_ref[pl.ds(lo, d), :], ht_ref[...], (((1,), (0,)), ((), ())),
            preferred_element_type=jnp.float32).astype(jnp.bfloat16)
    part(0, _D, o0_ref)
    part(_D, 256, o1_ref)
    part(_D + 256, 64, o2_ref)
    o0_ref[pl.ds(_D, 1), :] = jnp.ones((1, _N), jnp.bfloat16)
    o1_ref[pl.ds(256, 1), :] = jnp.ones((1, _N), jnp.bfloat16)
    o2_ref[pl.ds(64, 64), :] = o2_ref[pl.ds(0, 64), :]
    o2_ref[pl.ds(128, 1), :] = jnp.ones((1, _N), jnp.bfloat16)


def _lse_kernel(hpt_ref, w_ref, b_ref, lse_ref, m_sc, s_sc,
                *, tile, chunk, nsteps):
    i = pl.program_id(0)
    nch = tile // chunk
    m_parts, s_parts = [], []
    for c in range(nch):
        rows = pl.ds(c * chunk, chunk)
        lt = jax.lax.dot_general(
            w_ref[rows, :].astype(jnp.bfloat16), hpt_ref[...],
            (((1,), (0,)), ((), ())),
            preferred_element_type=jnp.float32).astype(jnp.bfloat16)
        lt = lt + b_ref[rows, :].astype(jnp.bfloat16)
        m_c = jnp.max(lt, axis=0, keepdims=True)
        p = jnp.exp(lt - m_c)
        s_parts.append(jnp.sum(p, axis=0, keepdims=True,
                               dtype=jnp.float32))
        m_parts.append(m_c.astype(jnp.float32))
    m_sc[i] = jnp.concatenate(m_parts, axis=0)
    s_sc[i] = jnp.concatenate(s_parts, axis=0)

    @pl.when(i == nsteps - 1)
    def _fin():
        m = m_sc[...]
        mm = jnp.max(m, axis=(0, 1), keepdims=True)
        s = jnp.sum(s_sc[...] * jnp.exp(m - mm), axis=(0, 1),
                    keepdims=True)
        lse_ref[...] = (jnp.log(s) + mm).reshape(1, _N)


def _gather_body(w0, w1, w2, br0, br1, br2, i0, i1, i2, j0, j1, j2,
                 r0, r1, r2, s0, s1, s2,
                 idx_v, buf0, buf1, buf2, bufb, sem):
    wid = jax.lax.axis_index("s") * _NC + jax.lax.axis_index("c")
    base = wid * _BPW

    def one(tbl, idx_hbm, buf, out):
        pltpu.sync_copy(idx_hbm.at[pl.ds(base, _BPW)], idx_v)
        pltpu.async_copy(tbl.at[idx_v], buf, sem).wait()
        pltpu.sync_copy(buf, out.at[pl.ds(base, _BPW)])

    one(w0, i0, buf0, r0)
    one(w1, i1, buf1, r1)
    one(w2, i2, buf2, r2)
    one(br0, j0, bufb, s0)
    one(br1, j1, bufb, s1)
    one(br2, j2, bufb, s2)


def _combine_kernel(t_ref, m0_ref, m1_ref, m2_ref, r0_ref, r1_ref, r2_ref,
                    s0_ref, s1_ref, s2_ref, o0_ref, o1_ref, o2_ref,
                    lh_ref, l1_ref, l2_ref, cw_ref, cb_ref, o_ref):
    t = t_ref[...]
    io128 = jax.lax.broadcasted_iota(jnp.int32, (1, 128), 1)
    lane = jax.lax.broadcasted_iota(jnp.int32, (128, 128), 1)
    diag = (jax.lax.broadcasted_iota(jnp.int32, (128, 128), 0) == lane)

    def gdiag(r_ref, s_ref, m_ref, hb_ref, pair):
        parts = []
        for blk in range(_N // 128):
            sl = pl.ds(blk * 128, 128)
            m = m_ref[sl, :]
            bcol = jnp.sum(
                jnp.where(io128 == m, s_ref[sl, :], 0.0),
                axis=1, keepdims=True)
            r = r_ref[sl, :]
            if pair:
                r = jnp.where(lane // 64 == m % 2, r, 0.0)
            rb = jnp.concatenate(
                [r.astype(jnp.bfloat16),
                 bcol.astype(jnp.bfloat16)], axis=1)
            dm = jax.lax.dot_general(
                rb, hb_ref[:, sl], (((1,), (0,)), ((), ())),
                preferred_element_type=jnp.float32)
            parts.append(jnp.sum(jnp.where(diag, dm, 0.0), axis=0,
                                 keepdims=True))
        return jnp.concatenate(parts, axis=1)

    g0 = gdiag(r0_ref, s0_ref, m0_ref, o0_ref, False)
    g1 = gdiag(r1_ref, s1_ref, m1_ref, o1_ref, False)
    g2 = gdiag(r2_ref, s2_ref, m2_ref, o2_ref, True)

    cl = jax.lax.dot_general(
        cw_ref[...], o0_ref[pl.ds(0, _D), :], (((1,), (0,)), ((), ())),
        preferred_element_type=jnp.float32) + cb_ref[...]
    cl0 = cl[0:1, :]
    cl1 = cl[1:2, :]
    lh = lh_ref[...]
    m = jnp.maximum(jnp.maximum(lh, cl0), cl1)
    lse = m + jnp.log(jnp.exp(lh - m) + jnp.exp(cl0 - m) + jnp.exp(cl1 - m))
    in1 = (t >= _CUT1) & (t < _CUT2)
    in2 = t >= _CUT2
    g = jnp.where(in1, cl1, jnp.where(in2, cl0, g0))
    nll = lse - g
    nll = nll + jnp.where(in1, l1_ref[...] - g1, 0.0)
    nll = nll + jnp.where(in2, l2_ref[...] - g2, 0.0)
    o_ref[...] = nll


def _stream_lse(hpt, w, b, tile, chunk):
    v, d = w.shape
    nsteps = v // tile
    nch = tile // chunk
    part = pltpu.VMEM((nsteps, nch, _N), jnp.float32)
    return pl.pallas_call(
        functools.partial(_lse_kernel, tile=tile, chunk=chunk,
                          nsteps=nsteps),
        grid=(nsteps,),
        in_specs=[
            pl.BlockSpec((d, _N), lambda i: (0, 0)),
            pl.BlockSpec((tile, d), lambda i: (i, 0)),
            pl.BlockSpec((tile, 1), lambda i: (i, 0)),
        ],
        out_specs=pl.BlockSpec((1, _N), lambda i: (0, 0)),
        out_shape=jax.ShapeDtypeStruct((1, _N), jnp.float32),
        scratch_shapes=[part, part],
    )(hpt, w, b)


def kernel(hidden, target, cluster_weight, cluster_bias, proj0, proj1,
           proj2, w0, w1, w2, b0, b1, b2):
    bf = jnp.bfloat16
    f32 = jnp.float32

    pt = jnp.concatenate([proj0, proj1, proj2], axis=1).T.astype(bf)
    ht = hidden.T.astype(bf)
    w0f, w1f, w2f = (x.astype(f32) for x in (w0, w1, w2))
    b0f, b1f, b2f = (x.astype(f32).reshape(-1, 1) for x in (b0, b1, b2))

    hpt0, hpt1, hpt2 = pl.pallas_call(
        _proj_kernel,
        out_shape=[jax.ShapeDtypeStruct((_D + 1, _N), bf),
                   jax.ShapeDtypeStruct((257, _N), bf),
                   jax.ShapeDtypeStruct((129, _N), bf)],
    )(pt, ht)

    t = target.astype(jnp.int32).reshape(1, _N)
    i0 = jnp.clip(t, 0, _CUT1 - 1).reshape(_N)
    i1 = jnp.clip(t - _CUT1, 0, _CUT2 - _CUT1 - 1).reshape(_N)
    i2 = jnp.clip(t - _CUT2, 0, 100000 - _CUT2 - 1).reshape(_N)

    def bias128(b):
        n = b.shape[0]
        p = (-n) % 128
        return jnp.pad(b.reshape(-1), (0, p)).reshape(-1, 128)

    mesh = plsc.VectorSubcoreMesh(core_axis_name="c", subcore_axis_name="s")
    sc = pl.kernel(
        _gather_body, mesh=mesh,
        out_type=[jax.ShapeDtypeStruct((_N, _D), f32),
                  jax.ShapeDtypeStruct((_N, 256), f32),
                  jax.ShapeDtypeStruct((_N, 128), f32),
                  jax.ShapeDtypeStruct((_N, 128), f32),
                  jax.ShapeDtypeStruct((_N, 128), f32),
                  jax.ShapeDtypeStruct((_N, 128), f32)],
        scratch_types=[pltpu.VMEM((_BPW,), jnp.int32),
                       pltpu.VMEM((_BPW, _D), f32),
                       pltpu.VMEM((_BPW, 256), f32),
                       pltpu.VMEM((_BPW, 128), f32),
                       pltpu.VMEM((_BPW, 128), f32),
                       pltpu.SemaphoreType.DMA],
    )
    rows0, rows1, rows2, bs0, bs1, bs2 = sc(
        w0f, w1f, w2f.reshape(-1, 128),
        bias128(b0f), bias128(b1f), bias128(b2f),
        i0, i1, i2 // 2, i0 // 128, i1 // 128, i2 // 128)

    lse_h = _stream_lse(hpt0, w0f, b0f, 2000, 400)
    lse_1 = _stream_lse(hpt1, w1f, b1f, 2000, 400)
    lse_2 = _stream_lse(hpt2, w2f, b2f, 4000, 400)

    nll = pl.pallas_call(
        _combine_kernel,
        out_shape=jax.ShapeDtypeStruct((1, _N), jnp.float32),
    )(t, (i0 % 128).reshape(_N, 1), (i1 % 128).reshape(_N, 1),
      (i2 % 128).reshape(_N, 1), rows0, rows1, rows2, bs0, bs1, bs2,
      hpt0, hpt1, hpt2, lse_h, lse_1, lse_2,
      cluster_weight.astype(bf), cluster_bias.reshape(2, 1))
    return nll.reshape(_N)


# speedup vs baseline: 1.0001x; 1.0001x over previous
import functools

import jax
import jax.numpy as jnp
from jax.experimental import pallas as pl
from jax.experimental.pallas import tpu as pltpu
from jax.experimental.pallas import tpu_sc as plsc

_N = 2048
_D = 1024
_CUT1 = 20000
_CUT2 = 40000

_SC = plsc.get_sparse_core_info()
_NC, _NS, _L = _SC.num_cores, _SC.num_subcores, _SC.num_lanes
_NW = _NC * _NS
_BPW = _N // _NW


def _proj_kernel(pt_ref, ht_ref, o0_ref, o1_ref, o2_ref):
    def part(lo, d, o_ref):
        o_ref[pl.ds(0, d), :] = jax.lax.dot_general(
            pt_ref[pl.ds(lo, d), :], ht_ref[...], (((1,), (0,)), ((), ())),
            preferred_element_type=jnp.float32).astype(jnp.bfloat16)
    part(0, _D, o0_ref)
    part(_D, 256, o1_ref)
    part(_D + 256, 64, o2_ref)
    o0_ref[pl.ds(_D, 1), :] = jnp.ones((1, _N), jnp.bfloat16)
    o1_ref[pl.ds(256, 1), :] = jnp.ones((1, _N), jnp.bfloat16)
    o2_ref[pl.ds(64, 64), :] = o2_ref[pl.ds(0, 64), :]
    o2_ref[pl.ds(128, 1), :] = jnp.ones((1, _N), jnp.bfloat16)


def _lse_kernel(hpt_ref, w_ref, b_ref, lse_ref, m_sc, s_sc,
                *, tile, chunk, nsteps):
    i = pl.program_id(0)
    nch = tile // chunk
    m_parts, s_parts = [], []
    for c in range(nch):
        rows = pl.ds(c * chunk, chunk)
        lt = jax.lax.dot_general(
            w_ref[rows, :].astype(jnp.bfloat16), hpt_ref[...],
            (((1,), (0,)), ((), ())),
            preferred_element_type=jnp.float32).astype(jnp.bfloat16)
        lt = lt + b_ref[rows, :].astype(jnp.bfloat16)
        m_c = jnp.max(lt, axis=0, keepdims=True)
        p = jnp.exp(lt - m_c)
        s_parts.append(jnp.sum(p, axis=0, keepdims=True,
                               dtype=jnp.float32))
        m_parts.append(m_c.astype(jnp.float32))
    m_sc[i] = jnp.concatenate(m_parts, axis=0)
    s_sc[i] = jnp.concatenate(s_parts, axis=0)

    @pl.when(i == nsteps - 1)
    def _fin():
        m = m_sc[...]
        mm = jnp.max(m, axis=(0, 1), keepdims=True)
        s = jnp.sum(s_sc[...] * jnp.exp(m - mm), axis=(0, 1),
                    keepdims=True)
        lse_ref[...] = (jnp.log(s) + mm).reshape(1, _N)


def _gather_body(w0, w1, w2, br0, br1, br2, i0, i1, i2, j0, j1, j2,
                 r0, r1, r2, s0, s1, s2,
                 iv0, iv1, iv2, jv0, jv1, jv2,
                 buf0, buf1, buf2, bb0, bb1, bb2,
                 sm0, sm1, sm2, sn0, sn1, sn2):
    wid = jax.lax.axis_index("s") * _NC + jax.lax.axis_index("c")
    sl = pl.ds(wid * _BPW, _BPW)
    ivs = (iv0, iv1, iv2, jv0, jv1, jv2)
    for ih, iv in zip((i0, i1, i2, j0, j1, j2), ivs):
        pltpu.sync_copy(ih.at[sl], iv)
    bufs = (buf0, buf1, buf2, bb0, bb1, bb2)
    cps = [pltpu.async_copy(tbl.at[iv], buf, sem)
           for tbl, iv, buf, sem in zip(
               (w0, w1, w2, br0, br1, br2), ivs, bufs,
               (sm0, sm1, sm2, sn0, sn1, sn2))]
    for cp, buf, out in zip(cps, bufs, (r0, r1, r2, s0, s1, s2)):
        cp.wait()
        pltpu.sync_copy(buf, out.at[sl])


def _combine_kernel(t_ref, m0_ref, m1_ref, m2_ref, r0_ref, r1_ref, r2_ref,
                    s0_ref, s1_ref, s2_ref, o0_ref, o1_ref, o2_ref,
                    lh_ref, l1_ref, l2_ref, cw_ref, cb_ref, o_ref):
    t = t_ref[...]
    io128 = jax.lax.broadcasted_iota(jnp.int32, (1, 128), 1)
    lane = jax.lax.broadcasted_iota(jnp.int32, (128, 128), 1)
    diag = (jax.lax.broadcasted_iota(jnp.int32, (128, 128), 0) == lane)

    def gdiag(r_ref, s_ref, m_ref, hb_ref, pair):
        parts = []
        for blk in range(_N // 128):
            sl = pl.ds(blk * 128, 128)
            m = m_ref[sl, :]
            bcol = jnp.sum(
                jnp.where(io128 == m, s_ref[sl, :], 0.0),
                axis=1, keepdims=True)
            r = r_ref[sl, :]
            if pair:
                r = jnp.where(lane // 64 == m % 2, r, 0.0)
            rb = jnp.concatenate(
                [r.astype(jnp.bfloat16),
                 bcol.astype(jnp.bfloat16)], axis=1)
            dm = jax.lax.dot_general(
                rb, hb_ref[:, sl], (((1,), (0,)), ((), ())),
                preferred_element_type=jnp.float32)
            parts.append(jnp.sum(jnp.where(diag, dm, 0.0), axis=0,
                                 keepdims=True))
        return jnp.concatenate(parts, axis=1)

    g0 = gdiag(r0_ref, s0_ref, m0_ref, o0_ref, False)
    g1 = gdiag(r1_ref, s1_ref, m1_ref, o1_ref, False)
    g2 = gdiag(r2_ref, s2_ref, m2_ref, o2_ref, True)

    cl = jax.lax.dot_general(
        cw_ref[...], o0_ref[pl.ds(0, _D), :], (((1,), (0,)), ((), ())),
        preferred_element_type=jnp.float32) + cb_ref[...]
    cl0 = cl[0:1, :]
    cl1 = cl[1:2, :]
    lh = lh_ref[...]
    m = jnp.maximum(jnp.maximum(lh, cl0), cl1)
    lse = m + jnp.log(jnp.exp(lh - m) + jnp.exp(cl0 - m) + jnp.exp(cl1 - m))
    in1 = (t >= _CUT1) & (t < _CUT2)
    in2 = t >= _CUT2
    g = jnp.where(in1, cl1, jnp.where(in2, cl0, g0))
    nll = lse - g
    nll = nll + jnp.where(in1, l1_ref[...] - g1, 0.0)
    nll = nll + jnp.where(in2, l2_ref[...] - g2, 0.0)
    o_ref[...] = nll


def _stream_lse(hpt, w, b, tile, chunk):
    v, d = w.shape
    nsteps = v // tile
    nch = tile // chunk
    part = pltpu.VMEM((nsteps, nch, _N), jnp.float32)
    return pl.pallas_call(
        functools.partial(_lse_kernel, tile=tile, chunk=chunk,
                          nsteps=nsteps),
        grid=(nsteps,),
        in_specs=[
            pl.BlockSpec((d, _N), lambda i: (0, 0)),
            pl.BlockSpec((tile, d), lambda i: (i, 0)),
            pl.BlockSpec((tile, 1), lambda i: (i, 0)),
        ],
        out_specs=pl.BlockSpec((1, _N), lambda i: (0, 0)),
        out_shape=jax.ShapeDtypeStruct((1, _N), jnp.float32),
        scratch_shapes=[part, part],
    )(hpt, w, b)


def kernel(hidden, target, cluster_weight, cluster_bias, proj0, proj1,
           proj2, w0, w1, w2, b0, b1, b2):
    bf = jnp.bfloat16
    f32 = jnp.float32

    pt = jnp.concatenate([proj0, proj1, proj2], axis=1).T.astype(bf)
    ht = hidden.T.astype(bf)
    w0f, w1f, w2f = (x.astype(f32) for x in (w0, w1, w2))
    b0f, b1f, b2f = (x.astype(f32).reshape(-1, 1) for x in (b0, b1, b2))

    hpt0, hpt1, hpt2 = pl.pallas_call(
        _proj_kernel,
        out_shape=[jax.ShapeDtypeStruct((_D + 1, _N), bf),
                   jax.ShapeDtypeStruct((257, _N), bf),
                   jax.ShapeDtypeStruct((129, _N), bf)],
    )(pt, ht)

    t = target.astype(jnp.int32).reshape(1, _N)
    i0 = jnp.clip(t, 0, _CUT1 - 1).reshape(_N)
    i1 = jnp.clip(t - _CUT1, 0, _CUT2 - _CUT1 - 1).reshape(_N)
    i2 = jnp.clip(t - _CUT2, 0, 100000 - _CUT2 - 1).reshape(_N)

    def bias128(b):
        n = b.shape[0]
        p = (-n) % 128
        return jnp.pad(b.reshape(-1), (0, p)).reshape(-1, 128)

    mesh = plsc.VectorSubcoreMesh(core_axis_name="c", subcore_axis_name="s")
    sc = pl.kernel(
        _gather_body, mesh=mesh,
        out_type=[jax.ShapeDtypeStruct((_N, _D), f32),
                  jax.ShapeDtypeStruct((_N, 256), f32),
                  jax.ShapeDtypeStruct((_N, 128), f32),
                  jax.ShapeDtypeStruct((_N, 128), f32),
                  jax.ShapeDtypeStruct((_N, 128), f32),
                  jax.ShapeDtypeStruct((_N, 128), f32)],
        scratch_types=([pltpu.VMEM((_BPW,), jnp.int32)] * 6
                       + [pltpu.VMEM((_BPW, _D), f32),
                          pltpu.VMEM((_BPW, 256), f32),
                          pltpu.VMEM((_BPW, 128), f32),
                          pltpu.VMEM((_BPW, 128), f32),
                          pltpu.VMEM((_BPW, 128), f32),
                          pltpu.VMEM((_BPW, 128), f32)]
                       + [pltpu.SemaphoreType.DMA] * 6),
    )
    rows0, rows1, rows2, bs0, bs1, bs2 = sc(
        w0f, w1f, w2f.reshape(-1, 128),
        bias128(b0f), bias128(b1f), bias128(b2f),
        i0, i1, i2 // 2, i0 // 128, i1 // 128, i2 // 128)

    lse_h = _stream_lse(hpt0, w0f, b0f, 2000, 400)
    lse_1 = _stream_lse(hpt1, w1f, b1f, 2000, 400)
    lse_2 = _stream_lse(hpt2, w2f, b2f, 4000, 400)

    nll = pl.pallas_call(
        _combine_kernel,
        out_shape=jax.ShapeDtypeStruct((1, _N), jnp.float32),
    )(t, (i0 % 128).reshape(_N, 1), (i1 % 128).reshape(_N, 1),
      (i2 % 128).reshape(_N, 1), rows0, rows1, rows2, bs0, bs1, bs2,
      hpt0, hpt1, hpt2, lse_h, lse_1, lse_2,
      cluster_weight.astype(bf), cluster_bias.reshape(2, 1))
    return nll.reshape(_N)


# speedup vs baseline: 1.0471x; 1.0471x over previous
import functools

import jax
import jax.numpy as jnp
from jax.experimental import pallas as pl
from jax.experimental.pallas import tpu as pltpu
from jax.experimental.pallas import tpu_sc as plsc

_N = 2048
_D = 1024
_CUT1 = 20000
_CUT2 = 40000

_SC = plsc.get_sparse_core_info()
_NC, _NS, _L = _SC.num_cores, _SC.num_subcores, _SC.num_lanes
_NW = _NC * _NS
_BPW = _N // _NW


def _proj_kernel(pt_ref, ht_ref, o0_ref, o1_ref, o2_ref):
    def part(lo, d, o_ref):
        o_ref[pl.ds(0, d), :] = jax.lax.dot_general(
            pt_ref[pl.ds(lo, d), :], ht_ref[...], (((1,), (0,)), ((), ())),
            preferred_element_type=jnp.float32).astype(jnp.bfloat16)
    part(0, _D, o0_ref)
    part(_D, 256, o1_ref)
    part(_D + 256, 64, o2_ref)
    o0_ref[pl.ds(_D, 1), :] = jnp.ones((1, _N), jnp.bfloat16)
    o1_ref[pl.ds(256, 1), :] = jnp.ones((1, _N), jnp.bfloat16)
    o2_ref[pl.ds(64, 64), :] = o2_ref[pl.ds(0, 64), :]
    o2_ref[pl.ds(128, 1), :] = jnp.ones((1, _N), jnp.bfloat16)


def _lse_kernel(hpt_ref, w_ref, b_ref, lse_ref, m_sc, s_sc,
                *, tile, chunk, nsteps):
    i = pl.program_id(0)
    nch = tile // chunk
    m_parts, s_parts = [], []
    for c in range(nch):
        rows = pl.ds(c * chunk, chunk)
        lt = jax.lax.dot_general(
            w_ref[rows, :].astype(jnp.bfloat16), hpt_ref[...],
            (((1,), (0,)), ((), ())),
            preferred_element_type=jnp.float32).astype(jnp.bfloat16)
        lt = lt + b_ref[rows, :].astype(jnp.bfloat16)
        m_c = jnp.max(lt, axis=0, keepdims=True)
        p = jnp.exp(lt - m_c)
        s_parts.append(jnp.sum(p, axis=0, keepdims=True,
                               dtype=jnp.float32))
        m_parts.append(m_c.astype(jnp.float32))
    m_sc[i] = jnp.concatenate(m_parts, axis=0)
    s_sc[i] = jnp.concatenate(s_parts, axis=0)

    @pl.when(i == nsteps - 1)
    def _fin():
        m = m_sc[...]
        mm = jnp.max(m, axis=(0, 1), keepdims=True)
        s = jnp.sum(s_sc[...] * jnp.exp(m - mm), axis=(0, 1),
                    keepdims=True)
        lse_ref[...] = (jnp.log(s) + mm).reshape(1, _N)


def _lse_head_kernel(hpt_ref, t_ref, w_ref, b_ref, lse_ref, g_ref,
                     m_sc, s_sc, *, tile, chunk, nsteps):
    i = pl.program_id(0)
    nch = tile // chunk
    m_parts, s_parts = [], []
    g_acc = jnp.zeros((1, _N), jnp.float32)
    for c in range(nch):
        rows = pl.ds(c * chunk, chunk)
        lt = jax.lax.dot_general(
            w_ref[rows, :].astype(jnp.bfloat16), hpt_ref[...],
            (((1,), (0,)), ((), ())),
            preferred_element_type=jnp.float32).astype(jnp.bfloat16)
        lt = lt + b_ref[rows, :].astype(jnp.bfloat16)
        m_c = jnp.max(lt, axis=0, keepdims=True)
        p = jnp.exp(lt - m_c)
        s_parts.append(jnp.sum(p, axis=0, keepdims=True,
                               dtype=jnp.float32))
        m_parts.append(m_c.astype(jnp.float32))
        rio = jax.lax.broadcasted_iota(jnp.int32, (chunk, _N), 0)
        hit = rio == t_ref[...] - (i * tile + c * chunk)
        g_acc = g_acc + jnp.sum(
            jnp.where(hit, lt.astype(jnp.float32), 0.0),
            axis=0, keepdims=True)
    m_sc[i] = jnp.concatenate(m_parts, axis=0)
    s_sc[i] = jnp.concatenate(s_parts, axis=0)

    @pl.when(i == 0)
    def _init():
        g_ref[...] = g_acc

    @pl.when(i > 0)
    def _acc():
        g_ref[...] += g_acc

    @pl.when(i == nsteps - 1)
    def _fin():
        m = m_sc[...]
        mm = jnp.max(m, axis=(0, 1), keepdims=True)
        s = jnp.sum(s_sc[...] * jnp.exp(m - mm), axis=(0, 1),
                    keepdims=True)
        lse_ref[...] = (jnp.log(s) + mm).reshape(1, _N)


def _gather_body(w1, w2, br1, br2, i1, i2, j1, j2,
                 r1, r2, s1, s2,
                 iv1, iv2, jv1, jv2, buf1, buf2, bb1, bb2,
                 sm1, sm2, sn1, sn2):
    wid = jax.lax.axis_index("s") * _NC + jax.lax.axis_index("c")
    sl = pl.ds(wid * _BPW, _BPW)
    ivs = (iv1, iv2, jv1, jv2)
    for ih, iv in zip((i1, i2, j1, j2), ivs):
        pltpu.sync_copy(ih.at[sl], iv)
    bufs = (buf1, buf2, bb1, bb2)
    cps = [pltpu.async_copy(tbl.at[iv], buf, sem)
           for tbl, iv, buf, sem in zip(
               (w1, w2, br1, br2), ivs, bufs, (sm1, sm2, sn1, sn2))]
    for cp, buf, out in zip(cps, bufs, (r1, r2, s1, s2)):
        cp.wait()
        pltpu.sync_copy(buf, out.at[sl])


def _combine_kernel(t_ref, m1_ref, m2_ref, r1_ref, r2_ref,
                    s1_ref, s2_ref, o0_ref, o1_ref, o2_ref,
                    lh_ref, l1_ref, l2_ref, g0_ref, cw_ref, cb_ref, o_ref):
    t = t_ref[...]
    io128 = jax.lax.broadcasted_iota(jnp.int32, (1, 128), 1)
    lane = jax.lax.broadcasted_iota(jnp.int32, (128, 128), 1)
    diag = (jax.lax.broadcasted_iota(jnp.int32, (128, 128), 0) == lane)

    def gdiag(r_ref, s_ref, m_ref, hb_ref, pair):
        parts = []
        for blk in range(_N // 128):
            sl = pl.ds(blk * 128, 128)
            m = m_ref[sl, :]
            bcol = jnp.sum(
                jnp.where(io128 == m, s_ref[sl, :], 0.0),
                axis=1, keepdims=True)
            r = r_ref[sl, :]
            if pair:
                r = jnp.where(lane // 64 == m % 2, r, 0.0)
            rb = jnp.concatenate(
                [r.astype(jnp.bfloat16),
                 bcol.astype(jnp.bfloat16)], axis=1)
            dm = jax.lax.dot_general(
                rb, hb_ref[:, sl], (((1,), (0,)), ((), ())),
                preferred_element_type=jnp.float32)
            parts.append(jnp.sum(jnp.where(diag, dm, 0.0), axis=0,
                                 keepdims=True))
        return jnp.concatenate(parts, axis=1)

    g0 = g0_ref[...]
    g1 = gdiag(r1_ref, s1_ref, m1_ref, o1_ref, False)
    g2 = gdiag(r2_ref, s2_ref, m2_ref, o2_ref, True)

    cl = jax.lax.dot_general(
        cw_ref[...], o0_ref[pl.ds(0, _D), :], (((1,), (0,)), ((), ())),
        preferred_element_type=jnp.float32) + cb_ref[...]
    cl0 = cl[0:1, :]
    cl1 = cl[1:2, :]
    lh = lh_ref[...]
    m = jnp.maximum(jnp.maximum(lh, cl0), cl1)
    lse = m + jnp.log(jnp.exp(lh - m) + jnp.exp(cl0 - m) + jnp.exp(cl1 - m))
    in1 = (t >= _CUT1) & (t < _CUT2)
    in2 = t >= _CUT2
    g = jnp.where(in1, cl1, jnp.where(in2, cl0, g0))
    nll = lse - g
    nll = nll + jnp.where(in1, l1_ref[...] - g1, 0.0)
    nll = nll + jnp.where(in2, l2_ref[...] - g2, 0.0)
    o_ref[...] = nll


def _stream_lse(hpt, w, b, tile, chunk):
    v, d = w.shape
    nsteps = v // tile
    nch = tile // chunk
    part = pltpu.VMEM((nsteps, nch, _N), jnp.float32)
    return pl.pallas_call(
        functools.partial(_lse_kernel, tile=tile, chunk=chunk,
                          nsteps=nsteps),
        grid=(nsteps,),
        in_specs=[
            pl.BlockSpec((d, _N), lambda i: (0, 0)),
            pl.BlockSpec((tile, d), lambda i: (i, 0)),
            pl.BlockSpec((tile, 1), lambda i: (i, 0)),
        ],
        out_specs=pl.BlockSpec((1, _N), lambda i: (0, 0)),
        out_shape=jax.ShapeDtypeStruct((1, _N), jnp.float32),
        scratch_shapes=[part, part],
    )(hpt, w, b)


def kernel(hidden, target, cluster_weight, cluster_bias, proj0, proj1,
           proj2, w0, w1, w2, b0, b1, b2):
    bf = jnp.bfloat16
    f32 = jnp.float32

    pt = jnp.concatenate([proj0, proj1, proj2], axis=1).T.astype(bf)
    ht = hidden.T.astype(bf)
    w0f, w1f, w2f = (x.astype(f32) for x in (w0, w1, w2))
    b0f, b1f, b2f = (x.astype(f32).reshape(-1, 1) for x in (b0, b1, b2))

    hpt0, hpt1, hpt2 = pl.pallas_call(
        _proj_kernel,
        out_shape=[jax.ShapeDtypeStruct((_D + 1, _N), bf),
                   jax.ShapeDtypeStruct((257, _N), bf),
                   jax.ShapeDtypeStruct((129, _N), bf)],
    )(pt, ht)

    t = target.astype(jnp.int32).reshape(1, _N)
    i0 = jnp.clip(t, 0, _CUT1 - 1).reshape(_N)
    i1 = jnp.clip(t - _CUT1, 0, _CUT2 - _CUT1 - 1).reshape(_N)
    i2 = jnp.clip(t - _CUT2, 0, 100000 - _CUT2 - 1).reshape(_N)

    def bias128(b):
        n = b.shape[0]
        p = (-n) % 128
        return jnp.pad(b.reshape(-1), (0, p)).reshape(-1, 128)

    mesh = plsc.VectorSubcoreMesh(core_axis_name="c", subcore_axis_name="s")
    sc = pl.kernel(
        _gather_body, mesh=mesh,
        out_type=[jax.ShapeDtypeStruct((_N, 256), f32),
                  jax.ShapeDtypeStruct((_N, 128), f32),
                  jax.ShapeDtypeStruct((_N, 128), f32),
                  jax.ShapeDtypeStruct((_N, 128), f32)],
        scratch_types=([pltpu.VMEM((_BPW,), jnp.int32)] * 4
                       + [pltpu.VMEM((_BPW, 256), f32),
                          pltpu.VMEM((_BPW, 128), f32),
                          pltpu.VMEM((_BPW, 128), f32),
                          pltpu.VMEM((_BPW, 128), f32)]
                       + [pltpu.SemaphoreType.DMA] * 4),
    )
    rows1, rows2, bs1, bs2 = sc(
        w1f, w2f.reshape(-1, 128), bias128(b1f), bias128(b2f),
        i1, i2 // 2, i1 // 128, i2 // 128)

    tile0, chunk0 = 2000, 400
    part0 = pltpu.VMEM((_CUT1 // tile0, tile0 // chunk0, _N), f32)
    lse_h, g0 = pl.pallas_call(
        functools.partial(_lse_head_kernel, tile=tile0, chunk=chunk0,
                          nsteps=_CUT1 // tile0),
        grid=(_CUT1 // tile0,),
        in_specs=[
            pl.BlockSpec((_D, _N), lambda i: (0, 0)),
            pl.BlockSpec((1, _N), lambda i: (0, 0)),
            pl.BlockSpec((tile0, _D), lambda i: (i, 0)),
            pl.BlockSpec((tile0, 1), lambda i: (i, 0)),
        ],
        out_specs=[pl.BlockSpec((1, _N), lambda i: (0, 0)),
                   pl.BlockSpec((1, _N), lambda i: (0, 0))],
        out_shape=[jax.ShapeDtypeStruct((1, _N), f32),
                   jax.ShapeDtypeStruct((1, _N), f32)],
        scratch_shapes=[part0, part0],
    )(hpt0, i0.reshape(1, _N), w0f, b0f)
    lse_1 = _stream_lse(hpt1, w1f, b1f, 2000, 400)
    lse_2 = _stream_lse(hpt2, w2f, b2f, 4000, 400)

    nll = pl.pallas_call(
        _combine_kernel,
        out_shape=jax.ShapeDtypeStruct((1, _N), jnp.float32),
    )(t, (i1 % 128).reshape(_N, 1), (i2 % 128).reshape(_N, 1),
      rows1, rows2, bs1, bs2,
      hpt0, hpt1, hpt2, lse_h, lse_1, lse_2, g0,
      cluster_weight.astype(bf), cluster_bias.reshape(2, 1))
    return nll.reshape(_N)


# speedup vs baseline: 1.0473x; 1.0002x over previous
import functools

import jax
import jax.numpy as jnp
from jax.experimental import pallas as pl
from jax.experimental.pallas import tpu as pltpu
from jax.experimental.pallas import tpu_sc as plsc

_N = 2048
_D = 1024
_CUT1 = 20000
_CUT2 = 40000

_SC = plsc.get_sparse_core_info()
_NC, _NS, _L = _SC.num_cores, _SC.num_subcores, _SC.num_lanes
_NW = _NC * _NS
_BPW = _N // _NW


def _proj_kernel(pt_ref, ht_ref, cw_ref, cb_ref, o0_ref, o1_ref, o2_ref,
                 cl_ref):
    def part(lo, d, o_ref):
        o_ref[pl.ds(0, d), :] = jax.lax.dot_general(
            pt_ref[pl.ds(lo, d), :], ht_ref[...], (((1,), (0,)), ((), ())),
            preferred_element_type=jnp.float32).astype(jnp.bfloat16)
    part(0, _D, o0_ref)
    part(_D, 256, o1_ref)
    part(_D + 256, 64, o2_ref)
    cl_ref[...] = jax.lax.dot_general(
        cw_ref[...], o0_ref[pl.ds(0, _D), :], (((1,), (0,)), ((), ())),
        preferred_element_type=jnp.float32) + cb_ref[...]
    o0_ref[pl.ds(_D, 1), :] = jnp.ones((1, _N), jnp.bfloat16)
    o1_ref[pl.ds(256, 1), :] = jnp.ones((1, _N), jnp.bfloat16)
    o2_ref[pl.ds(64, 64), :] = o2_ref[pl.ds(0, 64), :]
    o2_ref[pl.ds(128, 1), :] = jnp.ones((1, _N), jnp.bfloat16)


def _lse_kernel(hpt_ref, w_ref, b_ref, lse_ref, m_sc, s_sc,
                *, tile, chunk, nsteps):
    i = pl.program_id(0)
    nch = tile // chunk
    m_parts, s_parts = [], []
    for c in range(nch):
        rows = pl.ds(c * chunk, chunk)
        lt = jax.lax.dot_general(
            w_ref[rows, :].astype(jnp.bfloat16), hpt_ref[...],
            (((1,), (0,)), ((), ())),
            preferred_element_type=jnp.float32).astype(jnp.bfloat16)
        lt = lt + b_ref[rows, :].astype(jnp.bfloat16)
        m_c = jnp.max(lt, axis=0, keepdims=True)
        p = jnp.exp(lt - m_c)
        s_parts.append(jnp.sum(p, axis=0, keepdims=True,
                               dtype=jnp.float32))
        m_parts.append(m_c.astype(jnp.float32))
    m_sc[i] = jnp.concatenate(m_parts, axis=0)
    s_sc[i] = jnp.concatenate(s_parts, axis=0)

    @pl.when(i == nsteps - 1)
    def _fin():
        m = m_sc[...]
        mm = jnp.max(m, axis=(0, 1), keepdims=True)
        s = jnp.sum(s_sc[...] * jnp.exp(m - mm), axis=(0, 1),
                    keepdims=True)
        lse_ref[...] = (jnp.log(s) + mm).reshape(1, _N)


def _lse_head_kernel(hpt_ref, t_ref, w_ref, b_ref, lse_ref, g_ref,
                     m_sc, s_sc, *, tile, chunk, nsteps):
    i = pl.program_id(0)
    nch = tile // chunk
    m_parts, s_parts = [], []
    g_acc = jnp.zeros((1, _N), jnp.float32)
    for c in range(nch):
        rows = pl.ds(c * chunk, chunk)
        lt = jax.lax.dot_general(
            w_ref[rows, :].astype(jnp.bfloat16), hpt_ref[...],
            (((1,), (0,)), ((), ())),
            preferred_element_type=jnp.float32).astype(jnp.bfloat16)
        lt = lt + b_ref[rows, :].astype(jnp.bfloat16)
        m_c = jnp.max(lt, axis=0, keepdims=True)
        p = jnp.exp(lt - m_c)
        s_parts.append(jnp.sum(p, axis=0, keepdims=True,
                               dtype=jnp.float32))
        m_parts.append(m_c.astype(jnp.float32))
        rio = jax.lax.broadcasted_iota(jnp.int32, (chunk, _N), 0)
        hit = rio == t_ref[...] - (i * tile + c * chunk)
        g_acc = g_acc + jnp.sum(
            jnp.where(hit, lt.astype(jnp.float32), 0.0),
            axis=0, keepdims=True)
    m_sc[i] = jnp.concatenate(m_parts, axis=0)
    s_sc[i] = jnp.concatenate(s_parts, axis=0)

    @pl.when(i == 0)
    def _init():
        g_ref[...] = g_acc

    @pl.when(i > 0)
    def _acc():
        g_ref[...] += g_acc

    @pl.when(i == nsteps - 1)
    def _fin():
        m = m_sc[...]
        mm = jnp.max(m, axis=(0, 1), keepdims=True)
        s = jnp.sum(s_sc[...] * jnp.exp(m - mm), axis=(0, 1),
                    keepdims=True)
        lse_ref[...] = (jnp.log(s) + mm).reshape(1, _N)


def _gather_body(w1, w2, br1, br2, i1, i2, j1, j2,
                 r1, r2, s1, s2,
                 iv1, iv2, jv1, jv2, buf1, buf2, bb1, bb2,
                 sm1, sm2, sn1, sn2):
    wid = jax.lax.axis_index("s") * _NC + jax.lax.axis_index("c")
    sl = pl.ds(wid * _BPW, _BPW)
    ivs = (iv1, iv2, jv1, jv2)
    for ih, iv in zip((i1, i2, j1, j2), ivs):
        pltpu.sync_copy(ih.at[sl], iv)
    bufs = (buf1, buf2, bb1, bb2)
    cps = [pltpu.async_copy(tbl.at[iv], buf, sem)
           for tbl, iv, buf, sem in zip(
               (w1, w2, br1, br2), ivs, bufs, (sm1, sm2, sn1, sn2))]
    for cp, buf, out in zip(cps, bufs, (r1, r2, s1, s2)):
        cp.wait()
        pltpu.sync_copy(buf, out.at[sl])


def _combine_kernel(t_ref, m1_ref, m2_ref, r1_ref, r2_ref,
                    s1_ref, s2_ref, o1_ref, o2_ref,
                    lh_ref, l1_ref, l2_ref, g0_ref, cl_ref, o_ref):
    t = t_ref[...]
    io128 = jax.lax.broadcasted_iota(jnp.int32, (1, 128), 1)
    lane = jax.lax.broadcasted_iota(jnp.int32, (128, 128), 1)
    diag = (jax.lax.broadcasted_iota(jnp.int32, (128, 128), 0) == lane)

    def gdiag(r_ref, s_ref, m_ref, hb_ref, pair):
        parts = []
        for blk in range(_N // 128):
            sl = pl.ds(blk * 128, 128)
            m = m_ref[sl, :]
            bcol = jnp.sum(
                jnp.where(io128 == m, s_ref[sl, :], 0.0),
                axis=1, keepdims=True)
            r = r_ref[sl, :]
            if pair:
                r = jnp.where(lane // 64 == m % 2, r, 0.0)
            rb = jnp.concatenate(
                [r.astype(jnp.bfloat16),
                 bcol.astype(jnp.bfloat16)], axis=1)
            dm = jax.lax.dot_general(
                rb, hb_ref[:, sl], (((1,), (0,)), ((), ())),
                preferred_element_type=jnp.float32)
            parts.append(jnp.sum(jnp.where(diag, dm, 0.0), axis=0,
                                 keepdims=True))
        return jnp.concatenate(parts, axis=1)

    g0 = g0_ref[...]
    g1 = gdiag(r1_ref, s1_ref, m1_ref, o1_ref, False)
    g2 = gdiag(r2_ref, s2_ref, m2_ref, o2_ref, True)

    cl0 = cl_ref[0:1, :]
    cl1 = cl_ref[1:2, :]
    lh = lh_ref[...]
    m = jnp.maximum(jnp.maximum(lh, cl0), cl1)
    lse = m + jnp.log(jnp.exp(lh - m) + jnp.exp(cl0 - m) + jnp.exp(cl1 - m))
    in1 = (t >= _CUT1) & (t < _CUT2)
    in2 = t >= _CUT2
    g = jnp.where(in1, cl1, jnp.where(in2, cl0, g0))
    nll = lse - g
    nll = nll + jnp.where(in1, l1_ref[...] - g1, 0.0)
    nll = nll + jnp.where(in2, l2_ref[...] - g2, 0.0)
    o_ref[...] = nll


def _stream_lse(hpt, w, b, tile, chunk):
    v, d = w.shape
    nsteps = v // tile
    nch = tile // chunk
    part = pltpu.VMEM((nsteps, nch, _N), jnp.float32)
    return pl.pallas_call(
        functools.partial(_lse_kernel, tile=tile, chunk=chunk,
                          nsteps=nsteps),
        grid=(nsteps,),
        in_specs=[
            pl.BlockSpec((d, _N), lambda i: (0, 0)),
            pl.BlockSpec((tile, d), lambda i: (i, 0)),
            pl.BlockSpec((tile, 1), lambda i: (i, 0)),
        ],
        out_specs=pl.BlockSpec((1, _N), lambda i: (0, 0)),
        out_shape=jax.ShapeDtypeStruct((1, _N), jnp.float32),
        scratch_shapes=[part, part],
    )(hpt, w, b)


def kernel(hidden, target, cluster_weight, cluster_bias, proj0, proj1,
           proj2, w0, w1, w2, b0, b1, b2):
    bf = jnp.bfloat16
    f32 = jnp.float32

    pt = jnp.concatenate([proj0, proj1, proj2], axis=1).T.astype(bf)
    ht = hidden.T.astype(bf)
    w0f, w1f, w2f = (x.astype(f32) for x in (w0, w1, w2))
    b0f, b1f, b2f = (x.astype(f32).reshape(-1, 1) for x in (b0, b1, b2))

    hpt0, hpt1, hpt2, cl = pl.pallas_call(
        _proj_kernel,
        out_shape=[jax.ShapeDtypeStruct((_D + 1, _N), bf),
                   jax.ShapeDtypeStruct((257, _N), bf),
                   jax.ShapeDtypeStruct((129, _N), bf),
                   jax.ShapeDtypeStruct((2, _N), jnp.float32)],
    )(pt, ht, cluster_weight.astype(bf), cluster_bias.reshape(2, 1))

    t = target.astype(jnp.int32).reshape(1, _N)
    i0 = jnp.clip(t, 0, _CUT1 - 1).reshape(_N)
    i1 = jnp.clip(t - _CUT1, 0, _CUT2 - _CUT1 - 1).reshape(_N)
    i2 = jnp.clip(t - _CUT2, 0, 100000 - _CUT2 - 1).reshape(_N)

    def bias128(b):
        n = b.shape[0]
        p = (-n) % 128
        return jnp.pad(b.reshape(-1), (0, p)).reshape(-1, 128)

    mesh = plsc.VectorSubcoreMesh(core_axis_name="c", subcore_axis_name="s")
    sc = pl.kernel(
        _gather_body, mesh=mesh,
        out_type=[jax.ShapeDtypeStruct((_N, 256), f32),
                  jax.ShapeDtypeStruct((_N, 128), f32),
                  jax.ShapeDtypeStruct((_N, 128), f32),
                  jax.ShapeDtypeStruct((_N, 128), f32)],
        scratch_types=([pltpu.VMEM((_BPW,), jnp.int32)] * 4
                       + [pltpu.VMEM((_BPW, 256), f32),
                          pltpu.VMEM((_BPW, 128), f32),
                          pltpu.VMEM((_BPW, 128), f32),
                          pltpu.VMEM((_BPW, 128), f32)]
                       + [pltpu.SemaphoreType.DMA] * 4),
    )
    rows1, rows2, bs1, bs2 = sc(
        w1f, w2f.reshape(-1, 128), bias128(b1f), bias128(b2f),
        i1, i2 // 2, i1 // 128, i2 // 128)

    tile0, chunk0 = 2000, 400
    part0 = pltpu.VMEM((_CUT1 // tile0, tile0 // chunk0, _N), f32)
    lse_h, g0 = pl.pallas_call(
        functools.partial(_lse_head_kernel, tile=tile0, chunk=chunk0,
                          nsteps=_CUT1 // tile0),
        grid=(_CUT1 // tile0,),
        in_specs=[
            pl.BlockSpec((_D, _N), lambda i: (0, 0)),
            pl.BlockSpec((1, _N), lambda i: (0, 0)),
            pl.BlockSpec((tile0, _D), lambda i: (i, 0)),
            pl.BlockSpec((tile0, 1), lambda i: (i, 0)),
        ],
        out_specs=[pl.BlockSpec((1, _N), lambda i: (0, 0)),
                   pl.BlockSpec((1, _N), lambda i: (0, 0))],
        out_shape=[jax.ShapeDtypeStruct((1, _N), f32),
                   jax.ShapeDtypeStruct((1, _N), f32)],
        scratch_shapes=[part0, part0],
    )(hpt0, i0.reshape(1, _N), w0f, b0f)
    lse_1 = _stream_lse(hpt1, w1f, b1f, 2000, 400)
    lse_2 = _stream_lse(hpt2, w2f, b2f, 4000, 400)

    nll = pl.pallas_call(
        _combine_kernel,
        out_shape=jax.ShapeDtypeStruct((1, _N), jnp.float32),
    )(t, (i1 % 128).reshape(_N, 1), (i2 % 128).reshape(_N, 1),
      rows1, rows2, bs1, bs2,
      hpt1, hpt2, lse_h, lse_1, lse_2, g0, cl)
    return nll.reshape(_N)
